# Initial kernel scaffold; baseline (speedup 1.0000x reference)
import jax
import jax.numpy as jnp
from jax.experimental import pallas as pl


def kernel(users, items, user_emb_gmf, item_emb_gmf, user_emb_mlp, item_emb_mlp, W1, b1, W2, b2, proj_w):
    raise NotImplementedError("write your pallas kernel here")


# speedup vs baseline: 2.1741x; 2.1741x over previous
import functools

import jax
import jax.numpy as jnp
from jax import lax
from jax.experimental import pallas as pl
from jax.experimental.pallas import tpu as pltpu
from jax.experimental.pallas import tpu_sc as plsc

BATCH = 16384
EMBED = 128
NC, NS = 2, 16
NW = NC * NS
B_PER_W = BATCH // NW
CHUNK = 128
NCHUNK = B_PER_W // CHUNK


def _sc_gather4(users2d, items2d, t_ug, t_ig, t_um, t_im):
    mesh = plsc.VectorSubcoreMesh(core_axis_name="c", subcore_axis_name="s")
    row_t = jax.ShapeDtypeStruct((BATCH, EMBED), jnp.float32)

    @functools.partial(
        pl.kernel,
        mesh=mesh,
        out_type=(row_t, row_t, row_t, row_t),
        scratch_types=[
            pltpu.VMEM((NCHUNK, CHUNK), jnp.int32),
            pltpu.VMEM((NCHUNK, CHUNK), jnp.int32),
            pltpu.VMEM((CHUNK, EMBED), jnp.float32),
            pltpu.SemaphoreType.DMA,
        ],
    )
    def k(u_ref, i_ref, ug_ref, ig_ref, um_ref, im_ref,
          o_ug, o_ig, o_um, o_im, uidx, iidx, buf, sem):
        wid = lax.axis_index("s") * NC + lax.axis_index("c")
        base = wid * B_PER_W
        pltpu.sync_copy(u_ref.at[pl.ds(wid * NCHUNK, NCHUNK)], uidx)
        pltpu.sync_copy(i_ref.at[pl.ds(wid * NCHUNK, NCHUNK)], iidx)
        for table, idx, out in ((ug_ref, uidx, o_ug), (ig_ref, iidx, o_ig),
                                (um_ref, uidx, o_um), (im_ref, iidx, o_im)):
            for j in range(NCHUNK):
                pltpu.async_copy(table.at[idx.at[j]], buf, sem).wait()
                pltpu.sync_copy(buf, out.at[pl.ds(base + j * CHUNK, CHUNK)])

    return k(users2d, items2d, t_ug, t_ig, t_um, t_im)


BLK = 1024


def _dense_body(ug, ig, um, im, w1, b1, w2, b2, pw, out):
    h = jnp.maximum(
        um[:] @ w1[0:EMBED, :] + im[:] @ w1[EMBED:2 * EMBED, :] + b1[:], 0.0)
    m = jnp.maximum(h @ w2[:] + b2[:], 0.0)
    s = (ug[:] * ig[:]) @ pw[0:EMBED, :] + m @ pw[EMBED:EMBED + 64, :]
    out[:] = s


def _tc_dense(ug, ig, um, im, W1, b1, W2, b2, proj_w):
    grid = (BATCH // BLK,)
    row_spec = pl.BlockSpec((BLK, EMBED), lambda i: (i, 0))
    full = lambda shape: pl.BlockSpec(shape, lambda i: (0,) * len(shape))
    return pl.pallas_call(
        _dense_body,
        grid=grid,
        in_specs=[
            row_spec, row_spec, row_spec, row_spec,
            full((2 * EMBED, EMBED)), full((1, EMBED)),
            full((EMBED, 64)), full((1, 64)),
            full((EMBED + 64, 1)),
        ],
        out_specs=pl.BlockSpec((BLK, 1), lambda i: (i, 0)),
        out_shape=jax.ShapeDtypeStruct((BATCH, 1), jnp.float32),
    )(ug, ig, um, im, W1, b1.reshape(1, EMBED), W2, b2.reshape(1, 64), proj_w)


def kernel(users, items, user_emb_gmf, item_emb_gmf, user_emb_mlp,
           item_emb_mlp, W1, b1, W2, b2, proj_w):
    users2d = users.astype(jnp.int32).reshape(NW * NCHUNK, CHUNK)
    items2d = items.astype(jnp.int32).reshape(NW * NCHUNK, CHUNK)
    ug, ig, um, im = _sc_gather4(users2d, items2d, user_emb_gmf, item_emb_gmf,
                                 user_emb_mlp, item_emb_mlp)
    score = _tc_dense(ug, ig, um, im, W1, b1, W2, b2, proj_w)
    return score[:, 0]


# speedup vs baseline: 2.4733x; 1.1376x over previous
import functools

import jax
import jax.numpy as jnp
from jax import lax
from jax.experimental import pallas as pl
from jax.experimental.pallas import tpu as pltpu
from jax.experimental.pallas import tpu_sc as plsc

BATCH = 16384
EMBED = 128
NC, NS = 2, 16
NW = NC * NS
B_PER_W = BATCH // NW
CHUNK = 128
NCHUNK = B_PER_W // CHUNK


def _sc_gather4(users2d, items2d, t_ug, t_ig, t_um, t_im):
    mesh = plsc.VectorSubcoreMesh(core_axis_name="c", subcore_axis_name="s")
    row_t = jax.ShapeDtypeStruct((BATCH, EMBED), jnp.float32)

    nbuf = 4
    ntask = 4 * NCHUNK

    @functools.partial(
        pl.kernel,
        mesh=mesh,
        out_type=(row_t, row_t, row_t, row_t),
        scratch_types=[
            pltpu.VMEM((NCHUNK, CHUNK), jnp.int32),
            pltpu.VMEM((NCHUNK, CHUNK), jnp.int32),
            pltpu.VMEM((nbuf, CHUNK, EMBED), jnp.float32),
        ] + [pltpu.SemaphoreType.DMA] * (2 * nbuf),
    )
    def k(u_ref, i_ref, ug_ref, ig_ref, um_ref, im_ref,
          o_ug, o_ig, o_um, o_im, uidx, iidx, buf, *sems):
        gsem, ssem = sems[:nbuf], sems[nbuf:]
        wid = lax.axis_index("s") * NC + lax.axis_index("c")
        base = wid * B_PER_W
        pltpu.sync_copy(u_ref.at[pl.ds(wid * NCHUNK, NCHUNK)], uidx)
        pltpu.sync_copy(i_ref.at[pl.ds(wid * NCHUNK, NCHUNK)], iidx)
        tabs = (ug_ref, ig_ref, um_ref, im_ref)
        idxs = (uidx, iidx, uidx, iidx)
        outs = (o_ug, o_ig, o_um, o_im)
        tasks = [(tabs[t], idxs[t], outs[t], j)
                 for t in range(4) for j in range(NCHUNK)]

        def start_gather(kk):
            tb, ix, _, j = tasks[kk]
            b = kk % nbuf
            return pltpu.async_copy(tb.at[ix.at[j]], buf.at[b], gsem[b])

        gh = [start_gather(b) for b in range(nbuf)]
        sh = [None] * nbuf
        for kk in range(ntask):
            b = kk % nbuf
            gh[b].wait()
            _, _, out, j = tasks[kk]
            sh[b] = pltpu.async_copy(
                buf.at[b], out.at[pl.ds(base + j * CHUNK, CHUNK)], ssem[b])
            if kk + nbuf < ntask:
                sh[b].wait()
                gh[b] = start_gather(kk + nbuf)
        for kk in range(ntask - nbuf, ntask):
            sh[kk % nbuf].wait()

    return k(users2d, items2d, t_ug, t_ig, t_um, t_im)


BLK = 1024


def _dense_body(ug, ig, um, im, w1, b1, w2, b2, pw, out):
    h = jnp.maximum(
        um[:] @ w1[0:EMBED, :] + im[:] @ w1[EMBED:2 * EMBED, :] + b1[:], 0.0)
    m = jnp.maximum(h @ w2[:] + b2[:], 0.0)
    s = (ug[:] * ig[:]) @ pw[0:EMBED, :] + m @ pw[EMBED:EMBED + 64, :]
    out[:] = s


def _tc_dense(ug, ig, um, im, W1, b1, W2, b2, proj_w):
    grid = (BATCH // BLK,)
    row_spec = pl.BlockSpec((BLK, EMBED), lambda i: (i, 0))
    full = lambda shape: pl.BlockSpec(shape, lambda i: (0,) * len(shape))
    return pl.pallas_call(
        _dense_body,
        grid=grid,
        in_specs=[
            row_spec, row_spec, row_spec, row_spec,
            full((2 * EMBED, EMBED)), full((1, EMBED)),
            full((EMBED, 64)), full((1, 64)),
            full((EMBED + 64, 1)),
        ],
        out_specs=pl.BlockSpec((BLK, 1), lambda i: (i, 0)),
        out_shape=jax.ShapeDtypeStruct((BATCH, 1), jnp.float32),
    )(ug, ig, um, im, W1, b1.reshape(1, EMBED), W2, b2.reshape(1, 64), proj_w)


def kernel(users, items, user_emb_gmf, item_emb_gmf, user_emb_mlp,
           item_emb_mlp, W1, b1, W2, b2, proj_w):
    users2d = users.astype(jnp.int32).reshape(NW * NCHUNK, CHUNK)
    items2d = items.astype(jnp.int32).reshape(NW * NCHUNK, CHUNK)
    ug, ig, um, im = _sc_gather4(users2d, items2d, user_emb_gmf, item_emb_gmf,
                                 user_emb_mlp, item_emb_mlp)
    score = _tc_dense(ug, ig, um, im, W1, b1, W2, b2, proj_w)
    return score[:, 0]
